# baseline (speedup 1.0000x reference)
import functools

import jax
import jax.numpy as jnp
from jax import lax
from jax.experimental import pallas as pl
from jax.experimental.pallas import tpu as pltpu
from jax.experimental.pallas import tpu_sc as plsc

_NUM_CORES = 2
_NUM_SUBCORES = 16
_NUM_WORKERS = _NUM_CORES * _NUM_SUBCORES
_BPV = 2
_NSLOT = 8
_AHEAD = 4


def kernel(token_ids, weight):
    batch, seq = token_ids.shape
    dim = weight.shape[1]

    per_worker = batch // _NUM_WORKERS
    visits = per_worker // _BPV

    mesh = plsc.VectorSubcoreMesh(core_axis_name="c", subcore_axis_name="s")

    @functools.partial(
        pl.kernel,
        mesh=mesh,
        out_type=jax.ShapeDtypeStruct((batch, seq, dim), weight.dtype),
        scratch_types=[
            pltpu.VMEM((per_worker, seq), jnp.int32),
            pltpu.VMEM((_NSLOT, _BPV, seq, dim), jnp.float32),
            pltpu.SemaphoreType.DMA((_NSLOT,)),
            pltpu.SemaphoreType.DMA((_NSLOT,)),
        ],
        compiler_params=pltpu.CompilerParams(use_tc_tiling_on_sc=False),
    )
    def gather_kernel(table_hbm, idx_hbm, out_hbm, idx_v, rows_v, gsem, osem):
        wid = lax.axis_index("s") * _NUM_CORES + lax.axis_index("c")
        base = wid * per_worker
        pltpu.sync_copy(idx_hbm.at[pl.ds(base, per_worker)], idx_v)

        gather_d = {}
        out_d = {}

        def start_gathers(v):
            slot = v % _NSLOT
            gather_d[v] = [
                pltpu.async_copy(
                    table_hbm.at[idx_v.at[v * _BPV + k]],
                    rows_v.at[slot, k],
                    gsem.at[slot],
                )
                for k in range(_BPV)
            ]

        def start_out(v):
            slot = v % _NSLOT
            out_d[v] = pltpu.async_copy(
                rows_v.at[slot],
                out_hbm.at[pl.ds(base + v * _BPV, _BPV)],
                osem.at[slot],
            )

        for j in range(_AHEAD):
            start_gathers(j)
        for v in range(visits):
            j = v + _AHEAD
            if j < visits:
                if j >= _NSLOT:
                    out_d[j - _NSLOT].wait()
                start_gathers(j)
            for d in gather_d[v]:
                d.wait()
            start_out(v)
        for v in range(max(0, visits - _NSLOT), visits):
            out_d[v].wait()

    return gather_kernel(weight, token_ids.astype(jnp.int32))
